# baseline (speedup 1.0000x reference)
import functools
import math

import jax
import jax.numpy as jnp
from jax import lax
from jax.experimental import pallas as pl
from jax.experimental.pallas import tpu as pltpu
from jax.experimental.pallas import tpu_sc as plsc

NC = 2
NS = 16
NW = NC * NS
L = 16
D = 64
K = 128
NB = 5
WLAG = 4

N_U = 20000
N_I = 20000
N_B = 10000
BATCH = 4096


def _pad(n):
    m = 8 * NW
    return ((n + m - 1) // m) * m


N_UP = _pad(N_U)
N_IP = _pad(N_I)
N_BP = _pad(N_B)
MAXN = 20224

EGRAN = NW * K * 20


def _epad(e):
    return ((e + EGRAN - 1) // EGRAN) * EGRAN


_MESH = plsc.VectorSubcoreMesh(core_axis_name="c", subcore_axis_name="s")
_SC_PARAMS = pltpu.CompilerParams(use_tc_tiling_on_sc=False)


def _zero_fill(zbuf, rows, cols):
    zv = jnp.zeros((L,), jnp.float32)
    for i in range(rows):
        for j in range(cols // L):
            zbuf[i, pl.ds(j * L, L)] = zv


def _zero_region(zbuf, zrows, acc, base, total):
    full, r = divmod(total, zrows)
    for t in range(full):
        pltpu.sync_copy(zbuf, acc.at[pl.ds(base + t * zrows, zrows)])
    if r:
        pltpu.sync_copy(zbuf.at[pl.ds(0, r)],
                        acc.at[pl.ds(base + full * zrows, r)])


@functools.cache
def _deg_kernel(passes):
    NBD = 4
    ZR = 64

    out_types = []
    for (EP, napad, nbpad) in passes:
        out_types.append(jax.ShapeDtypeStruct((NC * napad, L), jnp.float32))
        out_types.append(jax.ShapeDtypeStruct((NC * nbpad, L), jnp.float32))

    @functools.partial(
        pl.kernel,
        out_type=tuple(out_types),
        mesh=_MESH,
        scratch_types=[
            pltpu.VMEM_SHARED((MAXN, L), jnp.float32),
            pltpu.VMEM_SHARED((MAXN, L), jnp.float32),
            pltpu.VMEM((ZR, L), jnp.float32),
            pltpu.VMEM((K, L), jnp.float32),
        ] + [pltpu.VMEM((K,), jnp.int32) for _ in range(2 * NBD)]
        + [pltpu.SemaphoreType.DMA for _ in range(4 * NBD)],
        compiler_params=_SC_PARAMS,
    )
    def body(*args):
        np_ = len(passes)
        ins = args[:2 * np_]
        outs = args[2 * np_:4 * np_]
        (acc_a, acc_b, zbuf, ones_v) = args[4 * np_:4 * np_ + 4]
        rest = args[4 * np_ + 4:]
        SI = rest[0:NBD]
        DI = rest[NBD:2 * NBD]
        sem_is = rest[2 * NBD:3 * NBD]
        sem_id = rest[3 * NBD:4 * NBD]
        sem_sa = rest[4 * NBD:5 * NBD]
        sem_sb = rest[5 * NBD:6 * NBD]
        c = lax.axis_index("c")
        s = lax.axis_index("s")
        wid = s * NC + c
        _zero_fill(zbuf, ZR, L)
        ov = jnp.full((L,), 1.0, jnp.float32)
        for i in range(K):
            ones_v[i, :] = ov

        for p, (EP, napad, nbpad) in enumerate(passes):
            src_hbm, dst_hbm = ins[2 * p], ins[2 * p + 1]
            outa, outb = outs[2 * p], outs[2 * p + 1]
            e_per = EP // NW
            n_full = e_per // K
            za = napad // NS
            zb = nbpad // NS
            _zero_region(zbuf, ZR, acc_a, s * za, za)
            _zero_region(zbuf, ZR, acc_b, s * zb, zb)
            plsc.subcore_barrier()

            def issue_idx(chunk, b):
                base = wid * e_per + chunk * K
                pltpu.async_copy(src_hbm.at[pl.ds(base, K)], SI[b],
                                 sem_is[b])
                pltpu.async_copy(dst_hbm.at[pl.ds(base, K)], DI[b],
                                 sem_id[b])

            def wait_idx(b):
                pltpu.make_async_copy(src_hbm.at[pl.ds(0, K)], SI[b],
                                      sem_is[b]).wait()
                pltpu.make_async_copy(dst_hbm.at[pl.ds(0, K)], DI[b],
                                      sem_id[b]).wait()

            def issue_scatter(b):
                pltpu.async_copy(ones_v, acc_a.at[SI[b]], sem_sa[b],
                                 add=True)
                pltpu.async_copy(ones_v, acc_b.at[DI[b]], sem_sb[b],
                                 add=True)

            def wait_scatter(b):
                pltpu.make_async_copy(outa.at[pl.ds(0, K)], ones_v,
                                      sem_sa[b]).wait()
                pltpu.make_async_copy(outa.at[pl.ds(0, K)], ones_v,
                                      sem_sb[b]).wait()

            issue_idx(0, 0)

            def outer(t, _):
                for kk in range(NBD):
                    chunk = NBD * t + kk
                    b = kk
                    bn = (kk + 1) % NBD
                    wait_idx(b)
                    issue_scatter(b)

                    @pl.when(chunk >= NBD - 1)
                    def _():
                        wait_scatter(bn)

                    @pl.when(chunk + 1 < n_full)
                    def _():
                        issue_idx(chunk + 1, bn)
                return 0

            lax.fori_loop(0, n_full // NBD, outer, 0)
            for chunk in range(n_full - (NBD - 1), n_full):
                wait_scatter(chunk % NBD)
            plsc.subcore_barrier()
            pltpu.sync_copy(acc_a.at[pl.ds(s * za, za)],
                            outa.at[pl.ds(c * napad + s * za, za)])
            pltpu.sync_copy(acc_b.at[pl.ds(s * zb, zb)],
                            outb.at[pl.ds(c * nbpad + s * zb, zb)])
            plsc.subcore_barrier()

    return body


@functools.cache
def _edge_kernel(passes):
    ZR = 64

    out_types = tuple(
        jax.ShapeDtypeStruct((NC * nd, D), jnp.float32)
        for (_, _, nd) in passes)

    @functools.partial(
        pl.kernel,
        out_type=out_types,
        mesh=_MESH,
        scratch_types=[
            pltpu.VMEM_SHARED((MAXN, D), jnp.float32),
            pltpu.VMEM((ZR, D), jnp.float32),
        ] + [pltpu.VMEM((K,), jnp.int32) for _ in range(2 * NB)]
        + [pltpu.VMEM((K, D), jnp.float32) for _ in range(NB)]
        + [pltpu.SemaphoreType.DMA for _ in range(4 * NB)],
        compiler_params=_SC_PARAMS,
    )
    def body(*args):
        np_ = len(passes)
        ins = args[:3 * np_]
        outs = args[3 * np_:4 * np_]
        acc, zbuf = args[4 * np_:4 * np_ + 2]
        rest = args[4 * np_ + 2:]
        GI = rest[0:NB]
        SI = rest[NB:2 * NB]
        RW = rest[2 * NB:3 * NB]
        sem_ig = rest[3 * NB:4 * NB]
        sem_is = rest[4 * NB:5 * NB]
        sem_g = rest[5 * NB:6 * NB]
        sem_s = rest[6 * NB:7 * NB]
        c = lax.axis_index("c")
        s = lax.axis_index("s")
        wid = s * NC + c
        _zero_fill(zbuf, ZR, D)

        for p, (EP, tabrows, ndstpad) in enumerate(passes):
            gidx_hbm, sidx_hbm, table_hbm = (ins[3 * p], ins[3 * p + 1],
                                             ins[3 * p + 2])
            out = outs[p]
            e_per = EP // NW
            n_full = e_per // K
            zr = ndstpad // NS
            _zero_region(zbuf, ZR, acc, s * zr, zr)
            plsc.subcore_barrier()

            def issue_idx(chunk, b):
                base = wid * e_per + chunk * K
                pltpu.async_copy(gidx_hbm.at[pl.ds(base, K)], GI[b],
                                 sem_ig[b])
                pltpu.async_copy(sidx_hbm.at[pl.ds(base, K)], SI[b],
                                 sem_is[b])

            def wait_idx(b):
                pltpu.make_async_copy(gidx_hbm.at[pl.ds(0, K)], GI[b],
                                      sem_ig[b]).wait()
                pltpu.make_async_copy(sidx_hbm.at[pl.ds(0, K)], SI[b],
                                      sem_is[b]).wait()

            def issue_gather(b):
                pltpu.async_copy(table_hbm.at[GI[b]], RW[b], sem_g[b])

            def wait_gather(b):
                pltpu.make_async_copy(table_hbm.at[pl.ds(0, K)], RW[b],
                                      sem_g[b]).wait()

            def issue_scatter(b):
                pltpu.async_copy(RW[b], acc.at[SI[b]], sem_s[b], add=True)

            def wait_scatter(b):
                pltpu.make_async_copy(table_hbm.at[pl.ds(0, K)], RW[b],
                                      sem_s[b]).wait()

            issue_idx(0, 0)

            def outer(t, _):
                for kk in range(NB):
                    chunk = NB * t + kk
                    b = kk
                    bn = (kk + 1) % NB
                    b2 = (kk - 2) % NB
                    bw = (kk - WLAG) % NB
                    wait_idx(b)
                    issue_gather(b)

                    @pl.when(chunk >= 2)
                    def _():
                        wait_gather(b2)
                        issue_scatter(b2)

                    @pl.when(chunk >= WLAG)
                    def _():
                        wait_scatter(bw)

                    @pl.when(chunk + 1 < n_full)
                    def _():
                        issue_idx(chunk + 1, bn)
                return 0

            lax.fori_loop(0, n_full // NB, outer, 0)
            for chunk in (n_full - 2, n_full - 1):
                wait_gather(chunk % NB)
                issue_scatter(chunk % NB)
            for chunk in range(n_full - WLAG, n_full):
                wait_scatter(chunk % NB)
            plsc.subcore_barrier()
            pltpu.sync_copy(acc.at[pl.ds(s * zr, zr)],
                            out.at[pl.ds(c * ndstpad + s * zr, zr)])
            plsc.subcore_barrier()

    return body


@functools.cache
def _batch_gather_kernel():
    b_per = BATCH // NW

    @functools.partial(
        pl.kernel,
        out_type=tuple(
            [jax.ShapeDtypeStruct((BATCH, D), jnp.float32)] * 6
            + [jax.ShapeDtypeStruct((BATCH, L), jnp.float32)] * 2),
        mesh=_MESH,
        scratch_types=[
            pltpu.VMEM((b_per,), jnp.int32),
            pltpu.VMEM((b_per,), jnp.int32),
            pltpu.VMEM((b_per, D), jnp.float32),
            pltpu.VMEM((b_per, D), jnp.float32),
            pltpu.VMEM((b_per, D), jnp.float32),
            pltpu.VMEM((b_per, D), jnp.float32),
            pltpu.VMEM((b_per, D), jnp.float32),
            pltpu.VMEM((b_per, D), jnp.float32),
            pltpu.VMEM((b_per, L), jnp.float32),
            pltpu.VMEM((b_per, L), jnp.float32),
            pltpu.SemaphoreType.DMA,
        ],
        compiler_params=_SC_PARAMS,
    )
    def body(uia, seg0, seg1, bib, uba, ubb, cnt0, cnt1, uidx, bidx,
             o_uui, o_s0, o_s1, o_bib, o_uub, o_ubb, o_c0, o_c1,
             ui_v, bi_v, b0, b1, b2, b3, b4, b5, b6, b7, sem):
        c = lax.axis_index("c")
        s = lax.axis_index("s")
        wid = s * NC + c
        base = wid * b_per
        pltpu.sync_copy(uidx.at[pl.ds(base, b_per)], ui_v)
        pltpu.sync_copy(bidx.at[pl.ds(base, b_per)], bi_v)
        cps = [
            pltpu.async_copy(uia.at[ui_v], b0, sem),
            pltpu.async_copy(seg0.at[bi_v], b1, sem),
            pltpu.async_copy(seg1.at[bi_v], b2, sem),
            pltpu.async_copy(bib.at[bi_v], b3, sem),
            pltpu.async_copy(uba.at[ui_v], b4, sem),
            pltpu.async_copy(ubb.at[bi_v], b5, sem),
            pltpu.async_copy(cnt0.at[bi_v], b6, sem),
            pltpu.async_copy(cnt1.at[bi_v], b7, sem),
        ]
        for cp in cps:
            cp.wait()
        for buf, out in ((b0, o_uui), (b1, o_s0), (b2, o_s1), (b3, o_bib),
                         (b4, o_uub), (b5, o_ubb), (b6, o_c0), (b7, o_c1)):
            pltpu.sync_copy(buf, out.at[pl.ds(base, b_per)])

    return body


_BLK = 256


def _isa_of(d0, d1):
    d = d0[:, :1] + d1[:, :1]
    return jnp.where(d > 0.0, lax.rsqrt(d), 0.0)


@functools.cache
def _fused_dense(kind, npad, nsets):
    n_in = {"pre": 3, "post": 5, "final": 5}[kind]
    n_out = {"pre": 1, "post": 2, "final": 1}[kind]

    def body(*refs):
        ins = refs[:n_in * nsets]
        outs = refs[n_in * nsets:]
        for t in range(nsets):
            a = ins[n_in * t:n_in * (t + 1)]
            o = outs[n_out * t:n_out * (t + 1)]
            if kind == "pre":
                d0, d1, x = a
                o[0][...] = x[...] * _isa_of(d0, d1)
            else:
                p0, p1, d0, d1, accin = a
                isa = _isa_of(d0, d1)
                x = isa * (p0[...] + p1[...])
                if kind == "final":
                    o[0][...] = (accin[...] + x) * jnp.float32(1.0 / 3.0)
                else:
                    o[0][...] = accin[...] + x
                    o[1][...] = isa * x

    row = lambda i: (i, 0)
    dspec = pl.BlockSpec((_BLK, L), row)
    xspec = pl.BlockSpec((_BLK, D), row)
    in_specs = {"pre": [dspec, dspec, xspec],
                "post": [xspec, xspec, dspec, dspec, xspec],
                "final": [xspec, xspec, dspec, dspec, xspec]}[kind] * nsets
    return pl.pallas_call(
        body,
        grid=(npad // _BLK,),
        in_specs=in_specs,
        out_specs=[xspec] * (n_out * nsets),
        out_shape=[jax.ShapeDtypeStruct((npad, D), jnp.float32)]
        * (n_out * nsets),
    )


@functools.cache
def _score_kernel():
    blk = 512
    lam = 1.0 / (1.0 + math.exp(-0.5))

    def body(uui, s0, s1, bib, uub, ubb, c0, c1, o):
        cnt = c0[:, :1] + c1[:, :1]
        b_items = (s0[...] + s1[...]) / (cnt + 1e-8)
        sc = jnp.sum(uui[...] * (b_items + jnp.float32(lam) * bib[...]),
                     axis=-1, keepdims=True)
        sc = sc + jnp.sum(uub[...] * ubb[...], axis=-1, keepdims=True)
        o[...] = sc

    row = lambda i: (i, 0)
    return pl.pallas_call(
        body,
        grid=(BATCH // blk,),
        in_specs=[pl.BlockSpec((blk, D), row)] * 6
        + [pl.BlockSpec((blk, L), row)] * 2,
        out_specs=pl.BlockSpec((blk, 1), row),
        out_shape=jax.ShapeDtypeStruct((BATCH, 1), jnp.float32),
    )


def _pad_rows(x, npad):
    return jnp.pad(x, ((0, npad - x.shape[0]), (0, 0)))


def _pad_edges(idx, ep, fill):
    return jnp.concatenate(
        [idx, jnp.full((ep - idx.shape[0],), fill, jnp.int32)])


def kernel(users_feature, items_feature, bundles_feature, ui_src, ui_dst,
           bi_src, bi_dst, ub_src, ub_dst, users_idx, bundles_idx):
    uf = _pad_rows(users_feature, N_UP)
    itf = _pad_rows(items_feature, N_IP)
    bf = _pad_rows(bundles_feature, N_BP)

    EP_UI = _epad(ui_src.shape[0])
    EP_BI = _epad(bi_src.shape[0])
    EP_UB = _epad(ub_src.shape[0])
    ui_s = _pad_edges(ui_src, EP_UI, N_UP - 1)
    ui_d = _pad_edges(ui_dst, EP_UI, N_IP - 1)
    bi_s = _pad_edges(bi_src, EP_BI, N_BP - 1)
    bi_d = _pad_edges(bi_dst, EP_BI, N_IP - 1)
    ub_s = _pad_edges(ub_src, EP_UB, N_UP - 1)
    ub_d = _pad_edges(ub_dst, EP_UB, N_BP - 1)

    dUIa, dUIb = _deg_kernel(((EP_UI, N_UP, N_IP),))(ui_s, ui_d)
    dBIa, dBIb = _deg_kernel(((EP_BI, N_BP, N_IP),))(bi_s, bi_d)
    dUBa, dUBb = _deg_kernel(((EP_UB, N_UP, N_BP),))(ub_s, ub_d)

    def hal(d, npad):
        return d[:npad], d[npad:]

    dUIa, dUIb = hal(dUIa, N_UP), hal(dUIb, N_IP)
    dBIa, dBIb = hal(dBIa, N_BP), hal(dBIb, N_IP)
    dUBa, dUBb = hal(dUBa, N_UP), hal(dUBb, N_BP)

    at0_UI, bt0_UI, bt0_BI, at0_UB = _fused_dense("pre", MAXN, 4)(
        *dUIa, uf, *dUIb, itf, *dBIb, itf, *dUBa, uf)
    at0_BI, bt0_UB = _fused_dense("pre", N_BP, 2)(*dBIa, bf, *dUBb, bf)

    PaUI = _edge_kernel(((EP_UI, N_IP, N_UP),))(ui_d, ui_s, bt0_UI)[0]
    PbUI = _edge_kernel(((EP_UI, N_UP, N_IP),))(ui_s, ui_d, at0_UI)[0]
    PaBI = _edge_kernel(((EP_BI, N_IP, N_BP),))(bi_d, bi_s, bt0_BI)[0]
    PbBI = _edge_kernel(((EP_BI, N_BP, N_IP),))(bi_s, bi_d, at0_BI)[0]
    PaUB = _edge_kernel(((EP_UB, N_BP, N_UP),))(ub_d, ub_s, bt0_UB)[0]
    PbUB = _edge_kernel(((EP_UB, N_UP, N_BP),))(ub_s, ub_d, at0_UB)[0]

    accA1_UI, at1_UI, accB1_UI, bt1_UI, accB1_BI, bt1_BI, accA1_UB, at1_UB = (
        _fused_dense("post", MAXN, 4)(
            *hal(PaUI, N_UP), *dUIa, uf,
            *hal(PbUI, N_IP), *dUIb, itf,
            *hal(PbBI, N_IP), *dBIb, itf,
            *hal(PaUB, N_UP), *dUBa, uf))
    accA1_BI, at1_BI, accB1_UB, bt1_UB = _fused_dense("post", N_BP, 2)(
        *hal(PaBI, N_BP), *dBIa, bf,
        *hal(PbUB, N_BP), *dUBb, bf)

    Pa2UI = _edge_kernel(((EP_UI, N_IP, N_UP),))(ui_d, ui_s, bt1_UI)[0]
    Pb2UI = _edge_kernel(((EP_UI, N_UP, N_IP),))(ui_s, ui_d, at1_UI)[0]
    Pa2BI = _edge_kernel(((EP_BI, N_IP, N_BP),))(bi_d, bi_s, bt1_BI)[0]
    Pa2UB = _edge_kernel(((EP_UB, N_BP, N_UP),))(ub_d, ub_s, bt1_UB)[0]
    Pb2UB = _edge_kernel(((EP_UB, N_UP, N_BP),))(ub_s, ub_d, at1_UB)[0]

    ui_u, ui_i, ub_u = _fused_dense("final", MAXN, 3)(
        *hal(Pa2UI, N_UP), *dUIa, accA1_UI,
        *hal(Pb2UI, N_IP), *dUIb, accB1_UI,
        *hal(Pa2UB, N_UP), *dUBa, accA1_UB)
    bi_b, ub_b = _fused_dense("final", N_BP, 2)(
        *hal(Pa2BI, N_BP), *dBIa, accA1_BI,
        *hal(Pb2UB, N_BP), *dUBb, accB1_UB)

    seg = _edge_kernel(((EP_BI, N_IP, N_BP),))(bi_d, bi_s, ui_i)[0]

    outs = _batch_gather_kernel()(
        ui_u, seg[:N_BP], seg[N_BP:], bi_b, ub_u, ub_b,
        dBIa[0], dBIa[1], users_idx, bundles_idx)
    score = _score_kernel()(*outs)
    return score[:, 0]


# speedup vs baseline: 2.6267x; 2.6267x over previous
import functools
import math

import jax
import jax.numpy as jnp
from jax import lax
from jax.experimental import pallas as pl
from jax.experimental.pallas import tpu as pltpu
from jax.experimental.pallas import tpu_sc as plsc

NC = 2
NS = 16
NW = NC * NS
L = 16
D = 64
K = 128
NB = 5
WLAG = 4

N_U = 20000
N_I = 20000
N_B = 10000
BATCH = 4096


def _pad(n):
    m = 8 * NW
    return ((n + m - 1) // m) * m


N_UP = _pad(N_U)
N_IP = _pad(N_I)
N_BP = _pad(N_B)
MAXN = 20224

EGRAN = NW * K * 20


def _epad(e):
    return ((e + EGRAN - 1) // EGRAN) * EGRAN


_MESH = plsc.VectorSubcoreMesh(core_axis_name="c", subcore_axis_name="s")
_SC_PARAMS = pltpu.CompilerParams(use_tc_tiling_on_sc=False)


def _zero_fill(zbuf, rows, cols):
    zv = jnp.zeros((L,), jnp.float32)
    for i in range(rows):
        for j in range(cols // L):
            zbuf[i, pl.ds(j * L, L)] = zv


def _zero_region(zbuf, zrows, acc, base, total):
    full, r = divmod(total, zrows)
    for t in range(full):
        pltpu.sync_copy(zbuf, acc.at[pl.ds(base + t * zrows, zrows)])
    if r:
        pltpu.sync_copy(zbuf.at[pl.ds(0, r)],
                        acc.at[pl.ds(base + full * zrows, r)])


@functools.cache
def _deg_kernel(passes):
    NBD = 4
    ZR = 64

    out_types = []
    for (EP, napad, nbpad) in passes:
        out_types.append(jax.ShapeDtypeStruct((NC * napad, L), jnp.float32))
        out_types.append(jax.ShapeDtypeStruct((NC * nbpad, L), jnp.float32))

    @functools.partial(
        pl.kernel,
        out_type=tuple(out_types),
        mesh=_MESH,
        scratch_types=[
            pltpu.VMEM_SHARED((MAXN, L), jnp.float32),
            pltpu.VMEM_SHARED((MAXN, L), jnp.float32),
            pltpu.VMEM((ZR, L), jnp.float32),
            pltpu.VMEM((K, L), jnp.float32),
        ] + [pltpu.VMEM((K,), jnp.int32) for _ in range(2 * NBD)]
        + [pltpu.SemaphoreType.DMA for _ in range(4 * NBD)],
        compiler_params=_SC_PARAMS,
    )
    def body(*args):
        np_ = len(passes)
        ins = args[:2 * np_]
        outs = args[2 * np_:4 * np_]
        (acc_a, acc_b, zbuf, ones_v) = args[4 * np_:4 * np_ + 4]
        rest = args[4 * np_ + 4:]
        SI = rest[0:NBD]
        DI = rest[NBD:2 * NBD]
        sem_is = rest[2 * NBD:3 * NBD]
        sem_id = rest[3 * NBD:4 * NBD]
        sem_sa = rest[4 * NBD:5 * NBD]
        sem_sb = rest[5 * NBD:6 * NBD]
        c = lax.axis_index("c")
        s = lax.axis_index("s")
        wid = s * NC + c
        _zero_fill(zbuf, ZR, L)
        ov = jnp.full((L,), 1.0, jnp.float32)
        for i in range(K):
            ones_v[i, :] = ov

        for p, (EP, napad, nbpad) in enumerate(passes):
            src_hbm, dst_hbm = ins[2 * p], ins[2 * p + 1]
            outa, outb = outs[2 * p], outs[2 * p + 1]
            e_per = EP // NW
            n_full = e_per // K
            za = napad // NS
            zb = nbpad // NS
            _zero_region(zbuf, ZR, acc_a, s * za, za)
            _zero_region(zbuf, ZR, acc_b, s * zb, zb)
            plsc.subcore_barrier()

            def issue_idx(chunk, b):
                base = wid * e_per + chunk * K
                pltpu.async_copy(src_hbm.at[pl.ds(base, K)], SI[b],
                                 sem_is[b])
                pltpu.async_copy(dst_hbm.at[pl.ds(base, K)], DI[b],
                                 sem_id[b])

            def wait_idx(b):
                pltpu.make_async_copy(src_hbm.at[pl.ds(0, K)], SI[b],
                                      sem_is[b]).wait()
                pltpu.make_async_copy(dst_hbm.at[pl.ds(0, K)], DI[b],
                                      sem_id[b]).wait()

            def issue_scatter(b):
                pltpu.async_copy(ones_v, acc_a.at[SI[b]], sem_sa[b],
                                 add=True)
                pltpu.async_copy(ones_v, acc_b.at[DI[b]], sem_sb[b],
                                 add=True)

            def wait_scatter(b):
                pltpu.make_async_copy(outa.at[pl.ds(0, K)], ones_v,
                                      sem_sa[b]).wait()
                pltpu.make_async_copy(outa.at[pl.ds(0, K)], ones_v,
                                      sem_sb[b]).wait()

            issue_idx(0, 0)

            def outer(t, _):
                for kk in range(NBD):
                    chunk = NBD * t + kk
                    b = kk
                    bn = (kk + 1) % NBD
                    wait_idx(b)
                    issue_scatter(b)

                    @pl.when(chunk >= NBD - 1)
                    def _():
                        wait_scatter(bn)

                    @pl.when(chunk + 1 < n_full)
                    def _():
                        issue_idx(chunk + 1, bn)
                return 0

            lax.fori_loop(0, n_full // NBD, outer, 0)
            for chunk in range(n_full - (NBD - 1), n_full):
                wait_scatter(chunk % NBD)
            plsc.subcore_barrier()
            pltpu.sync_copy(acc_a.at[pl.ds(s * za, za)],
                            outa.at[pl.ds(c * napad + s * za, za)])
            pltpu.sync_copy(acc_b.at[pl.ds(s * zb, zb)],
                            outb.at[pl.ds(c * nbpad + s * zb, zb)])
            plsc.subcore_barrier()

    return body


@functools.cache
def _edge_kernel(passes):
    ZR = 64

    out_types = tuple(
        jax.ShapeDtypeStruct((NC * nd, D), jnp.float32)
        for (_, _, nd) in passes)

    @functools.partial(
        pl.kernel,
        out_type=out_types,
        mesh=_MESH,
        scratch_types=[
            pltpu.VMEM_SHARED((MAXN, D), jnp.float32),
            pltpu.VMEM((ZR, D), jnp.float32),
        ] + [pltpu.VMEM((K,), jnp.int32) for _ in range(2 * NB)]
        + [pltpu.VMEM((K, D), jnp.float32) for _ in range(NB)]
        + [pltpu.SemaphoreType.DMA for _ in range(4 * NB)],
        compiler_params=_SC_PARAMS,
    )
    def body(*args):
        np_ = len(passes)
        ins = args[:3 * np_]
        outs = args[3 * np_:4 * np_]
        acc, zbuf = args[4 * np_:4 * np_ + 2]
        rest = args[4 * np_ + 2:]
        GI = rest[0:NB]
        SI = rest[NB:2 * NB]
        RW = rest[2 * NB:3 * NB]
        sem_ig = rest[3 * NB:4 * NB]
        sem_is = rest[4 * NB:5 * NB]
        sem_g = rest[5 * NB:6 * NB]
        sem_s = rest[6 * NB:7 * NB]
        c = lax.axis_index("c")
        s = lax.axis_index("s")
        wid = s * NC + c
        _zero_fill(zbuf, ZR, D)

        for p, (EP, tabrows, ndstpad) in enumerate(passes):
            gidx_hbm, sidx_hbm, table_hbm = (ins[3 * p], ins[3 * p + 1],
                                             ins[3 * p + 2])
            out = outs[p]
            e_per = EP // NW
            n_full = e_per // K
            zr = ndstpad // NS
            _zero_region(zbuf, ZR, acc, s * zr, zr)
            plsc.subcore_barrier()

            def issue_idx(chunk, b):
                base = wid * e_per + chunk * K
                pltpu.async_copy(gidx_hbm.at[pl.ds(base, K)], GI[b],
                                 sem_ig[b])
                pltpu.async_copy(sidx_hbm.at[pl.ds(base, K)], SI[b],
                                 sem_is[b])

            def wait_idx(b):
                pltpu.make_async_copy(gidx_hbm.at[pl.ds(0, K)], GI[b],
                                      sem_ig[b]).wait()
                pltpu.make_async_copy(sidx_hbm.at[pl.ds(0, K)], SI[b],
                                      sem_is[b]).wait()

            def issue_gather(b):
                pltpu.async_copy(table_hbm.at[GI[b]], RW[b], sem_g[b])

            def wait_gather(b):
                pltpu.make_async_copy(table_hbm.at[pl.ds(0, K)], RW[b],
                                      sem_g[b]).wait()

            def issue_scatter(b):
                pltpu.async_copy(RW[b], acc.at[SI[b]], sem_s[b], add=True)

            def wait_scatter(b):
                pltpu.make_async_copy(table_hbm.at[pl.ds(0, K)], RW[b],
                                      sem_s[b]).wait()

            issue_idx(0, 0)

            def outer(t, _):
                for kk in range(NB):
                    chunk = NB * t + kk
                    b = kk
                    bn = (kk + 1) % NB
                    b2 = (kk - 2) % NB
                    bw = (kk - WLAG) % NB
                    wait_idx(b)
                    issue_gather(b)

                    @pl.when(chunk >= 2)
                    def _():
                        wait_gather(b2)
                        issue_scatter(b2)

                    @pl.when(chunk >= WLAG)
                    def _():
                        wait_scatter(bw)

                    @pl.when(chunk + 1 < n_full)
                    def _():
                        issue_idx(chunk + 1, bn)
                return 0

            lax.fori_loop(0, n_full // NB, outer, 0)
            for chunk in (n_full - 2, n_full - 1):
                wait_gather(chunk % NB)
                issue_scatter(chunk % NB)
            for chunk in range(n_full - WLAG, n_full):
                wait_scatter(chunk % NB)
            plsc.subcore_barrier()
            pltpu.sync_copy(acc.at[pl.ds(s * zr, zr)],
                            out.at[pl.ds(c * ndstpad + s * zr, zr)])
            plsc.subcore_barrier()

    return body


@functools.cache
def _batch_gather_kernel():
    b_per = BATCH // NW

    @functools.partial(
        pl.kernel,
        out_type=tuple(
            [jax.ShapeDtypeStruct((BATCH, D), jnp.float32)] * 6
            + [jax.ShapeDtypeStruct((BATCH, L), jnp.float32)] * 2),
        mesh=_MESH,
        scratch_types=[
            pltpu.VMEM((b_per,), jnp.int32),
            pltpu.VMEM((b_per,), jnp.int32),
            pltpu.VMEM((b_per, D), jnp.float32),
            pltpu.VMEM((b_per, D), jnp.float32),
            pltpu.VMEM((b_per, D), jnp.float32),
            pltpu.VMEM((b_per, D), jnp.float32),
            pltpu.VMEM((b_per, D), jnp.float32),
            pltpu.VMEM((b_per, D), jnp.float32),
            pltpu.VMEM((b_per, L), jnp.float32),
            pltpu.VMEM((b_per, L), jnp.float32),
            pltpu.SemaphoreType.DMA,
        ],
        compiler_params=_SC_PARAMS,
    )
    def body(uia, seg0, seg1, bib, uba, ubb, cnt0, cnt1, uidx, bidx,
             o_uui, o_s0, o_s1, o_bib, o_uub, o_ubb, o_c0, o_c1,
             ui_v, bi_v, b0, b1, b2, b3, b4, b5, b6, b7, sem):
        c = lax.axis_index("c")
        s = lax.axis_index("s")
        wid = s * NC + c
        base = wid * b_per
        pltpu.sync_copy(uidx.at[pl.ds(base, b_per)], ui_v)
        pltpu.sync_copy(bidx.at[pl.ds(base, b_per)], bi_v)
        cps = [
            pltpu.async_copy(uia.at[ui_v], b0, sem),
            pltpu.async_copy(seg0.at[bi_v], b1, sem),
            pltpu.async_copy(seg1.at[bi_v], b2, sem),
            pltpu.async_copy(bib.at[bi_v], b3, sem),
            pltpu.async_copy(uba.at[ui_v], b4, sem),
            pltpu.async_copy(ubb.at[bi_v], b5, sem),
            pltpu.async_copy(cnt0.at[bi_v], b6, sem),
            pltpu.async_copy(cnt1.at[bi_v], b7, sem),
        ]
        for cp in cps:
            cp.wait()
        for buf, out in ((b0, o_uui), (b1, o_s0), (b2, o_s1), (b3, o_bib),
                         (b4, o_uub), (b5, o_ubb), (b6, o_c0), (b7, o_c1)):
            pltpu.sync_copy(buf, out.at[pl.ds(base, b_per)])

    return body


_BLK = 256


def _isa_of(d0, d1):
    d = d0[:, :1] + d1[:, :1]
    return jnp.where(d > 0.0, lax.rsqrt(d), 0.0)


@functools.cache
def _fused_dense(kind, npad, nsets):
    n_in = {"pre": 3, "post": 5, "final": 5}[kind]
    n_out = {"pre": 1, "post": 2, "final": 1}[kind]

    def body(*refs):
        ins = refs[:n_in * nsets]
        outs = refs[n_in * nsets:]
        for t in range(nsets):
            a = ins[n_in * t:n_in * (t + 1)]
            o = outs[n_out * t:n_out * (t + 1)]
            if kind == "pre":
                d0, d1, x = a
                o[0][...] = x[...] * _isa_of(d0, d1)
            else:
                p0, p1, d0, d1, accin = a
                isa = _isa_of(d0, d1)
                x = isa * (p0[...] + p1[...])
                if kind == "final":
                    o[0][...] = (accin[...] + x) * jnp.float32(1.0 / 3.0)
                else:
                    o[0][...] = accin[...] + x
                    o[1][...] = isa * x

    row = lambda i: (i, 0)
    dspec = pl.BlockSpec((_BLK, L), row)
    xspec = pl.BlockSpec((_BLK, D), row)
    in_specs = {"pre": [dspec, dspec, xspec],
                "post": [xspec, xspec, dspec, dspec, xspec],
                "final": [xspec, xspec, dspec, dspec, xspec]}[kind] * nsets
    return pl.pallas_call(
        body,
        grid=(npad // _BLK,),
        in_specs=in_specs,
        out_specs=[xspec] * (n_out * nsets),
        out_shape=[jax.ShapeDtypeStruct((npad, D), jnp.float32)]
        * (n_out * nsets),
    )


@functools.cache
def _score_kernel():
    blk = 512
    lam = 1.0 / (1.0 + math.exp(-0.5))

    def body(uui, s0, s1, bib, uub, ubb, c0, c1, o):
        cnt = c0[:, :1] + c1[:, :1]
        b_items = (s0[...] + s1[...]) / (cnt + 1e-8)
        sc = jnp.sum(uui[...] * (b_items + jnp.float32(lam) * bib[...]),
                     axis=-1, keepdims=True)
        sc = sc + jnp.sum(uub[...] * ubb[...], axis=-1, keepdims=True)
        o[...] = sc

    row = lambda i: (i, 0)
    return pl.pallas_call(
        body,
        grid=(BATCH // blk,),
        in_specs=[pl.BlockSpec((blk, D), row)] * 6
        + [pl.BlockSpec((blk, L), row)] * 2,
        out_specs=pl.BlockSpec((blk, 1), row),
        out_shape=jax.ShapeDtypeStruct((BATCH, 1), jnp.float32),
    )


def _pad_rows(x, npad):
    return jnp.pad(x, ((0, npad - x.shape[0]), (0, 0)))


def _pad_edges(idx, ep, lo, hi):
    n = ep - idx.shape[0]
    fill = lo + jnp.arange(n, dtype=jnp.int32) % (hi - lo)
    return jnp.concatenate([idx, fill])


def kernel(users_feature, items_feature, bundles_feature, ui_src, ui_dst,
           bi_src, bi_dst, ub_src, ub_dst, users_idx, bundles_idx):
    uf = _pad_rows(users_feature, N_UP)
    itf = _pad_rows(items_feature, N_IP)
    bf = _pad_rows(bundles_feature, N_BP)

    EP_UI = _epad(ui_src.shape[0])
    EP_BI = _epad(bi_src.shape[0])
    EP_UB = _epad(ub_src.shape[0])
    ui_s = _pad_edges(ui_src, EP_UI, N_U, N_UP)
    ui_d = _pad_edges(ui_dst, EP_UI, N_I, N_IP)
    bi_s = _pad_edges(bi_src, EP_BI, N_B, N_BP)
    bi_d = _pad_edges(bi_dst, EP_BI, N_I, N_IP)
    ub_s = _pad_edges(ub_src, EP_UB, N_U, N_UP)
    ub_d = _pad_edges(ub_dst, EP_UB, N_B, N_BP)

    dUIa, dUIb = _deg_kernel(((EP_UI, N_UP, N_IP),))(ui_s, ui_d)
    dBIa, dBIb = _deg_kernel(((EP_BI, N_BP, N_IP),))(bi_s, bi_d)
    dUBa, dUBb = _deg_kernel(((EP_UB, N_UP, N_BP),))(ub_s, ub_d)

    def hal(d, npad):
        return d[:npad], d[npad:]

    dUIa, dUIb = hal(dUIa, N_UP), hal(dUIb, N_IP)
    dBIa, dBIb = hal(dBIa, N_BP), hal(dBIb, N_IP)
    dUBa, dUBb = hal(dUBa, N_UP), hal(dUBb, N_BP)

    at0_UI, bt0_UI, bt0_BI, at0_UB = _fused_dense("pre", MAXN, 4)(
        *dUIa, uf, *dUIb, itf, *dBIb, itf, *dUBa, uf)
    at0_BI, bt0_UB = _fused_dense("pre", N_BP, 2)(*dBIa, bf, *dUBb, bf)

    PaUI = _edge_kernel(((EP_UI, N_IP, N_UP),))(ui_d, ui_s, bt0_UI)[0]
    PbUI = _edge_kernel(((EP_UI, N_UP, N_IP),))(ui_s, ui_d, at0_UI)[0]
    PaBI = _edge_kernel(((EP_BI, N_IP, N_BP),))(bi_d, bi_s, bt0_BI)[0]
    PbBI = _edge_kernel(((EP_BI, N_BP, N_IP),))(bi_s, bi_d, at0_BI)[0]
    PaUB = _edge_kernel(((EP_UB, N_BP, N_UP),))(ub_d, ub_s, bt0_UB)[0]
    PbUB = _edge_kernel(((EP_UB, N_UP, N_BP),))(ub_s, ub_d, at0_UB)[0]

    accA1_UI, at1_UI, accB1_UI, bt1_UI, accB1_BI, bt1_BI, accA1_UB, at1_UB = (
        _fused_dense("post", MAXN, 4)(
            *hal(PaUI, N_UP), *dUIa, uf,
            *hal(PbUI, N_IP), *dUIb, itf,
            *hal(PbBI, N_IP), *dBIb, itf,
            *hal(PaUB, N_UP), *dUBa, uf))
    accA1_BI, at1_BI, accB1_UB, bt1_UB = _fused_dense("post", N_BP, 2)(
        *hal(PaBI, N_BP), *dBIa, bf,
        *hal(PbUB, N_BP), *dUBb, bf)

    Pa2UI = _edge_kernel(((EP_UI, N_IP, N_UP),))(ui_d, ui_s, bt1_UI)[0]
    Pb2UI = _edge_kernel(((EP_UI, N_UP, N_IP),))(ui_s, ui_d, at1_UI)[0]
    Pa2BI = _edge_kernel(((EP_BI, N_IP, N_BP),))(bi_d, bi_s, bt1_BI)[0]
    Pa2UB = _edge_kernel(((EP_UB, N_BP, N_UP),))(ub_d, ub_s, bt1_UB)[0]
    Pb2UB = _edge_kernel(((EP_UB, N_UP, N_BP),))(ub_s, ub_d, at1_UB)[0]

    ui_u, ui_i, ub_u = _fused_dense("final", MAXN, 3)(
        *hal(Pa2UI, N_UP), *dUIa, accA1_UI,
        *hal(Pb2UI, N_IP), *dUIb, accB1_UI,
        *hal(Pa2UB, N_UP), *dUBa, accA1_UB)
    bi_b, ub_b = _fused_dense("final", N_BP, 2)(
        *hal(Pa2BI, N_BP), *dBIa, accA1_BI,
        *hal(Pb2UB, N_BP), *dUBb, accB1_UB)

    seg = _edge_kernel(((EP_BI, N_IP, N_BP),))(bi_d, bi_s, ui_i)[0]

    outs = _batch_gather_kernel()(
        ui_u, seg[:N_BP], seg[N_BP:], bi_b, ub_u, ub_b,
        dBIa[0], dBIa[1], users_idx, bundles_idx)
    score = _score_kernel()(*outs)
    return score[:, 0]
